# baseline (speedup 1.0000x reference)
import functools

import jax
import jax.numpy as jnp
from jax import lax
from jax.experimental import pallas as pl
from jax.experimental.pallas import tpu as pltpu
from jax.experimental.pallas import tpu_sc as plsc

_NUM_EXPERTS = 64
_TOP_K = 2
_ALPHA = 0.01
_TILE = 8192

_NC = 2
_NS = 16
_NW = _NC * _NS
_L = 16


def _matmul_body(x_ref, wt_ref, lt_out, ps_out, psacc, *, n_steps):
    pid = pl.program_id(0)

    @pl.when(pid == 0)
    def _init():
        psacc[...] = jnp.zeros_like(psacc)

    tile = x_ref.shape[0]
    logits = jnp.dot(x_ref[...], wt_ref[...],
                     preferred_element_type=jnp.float32)
    sub = tile // 1024
    lt_out[...] = logits.reshape(sub, 1024, _NUM_EXPERTS).transpose(0, 2, 1)

    m = jnp.max(logits, axis=-1, keepdims=True)
    e = jnp.exp(logits - m)
    z = jnp.sum(e, axis=-1, keepdims=True)
    psacc[...] += jnp.sum(e / z, axis=0, keepdims=True)

    @pl.when(pid == n_steps - 1)
    def _fin():
        ps_out[...] = psacc[...]


def _sc_router_body(lt_hbm, w_hbm, e_hbm, cnt_hbm,
                    blk, wstage, estage, cntacc, dsem):
    wid = lax.axis_index("s") * _NC + lax.axis_index("c")
    chunk = blk.shape[1]
    n_groups = chunk // _L

    cp = pltpu.async_copy(lt_hbm.at[wid], blk, dsem)

    zf = jnp.zeros((_L,), jnp.float32)

    def zero_body(e, _):
        cntacc[e, :] = zf
        return 0

    lax.fori_loop(0, _NUM_EXPERTS, zero_body, 0)
    cp.wait()

    def group_body(g, _):
        sl = pl.ds(g * _L, _L)

        m1 = blk[0, sl]
        i1 = jnp.zeros((_L,), jnp.int32)
        m2 = jnp.full((_L,), -jnp.inf, jnp.float32)
        i2 = jnp.zeros((_L,), jnp.int32)
        for e in range(1, _NUM_EXPERTS):
            v = blk[e, sl]
            ev = jnp.full((_L,), e, jnp.int32)
            gt1 = v > m1
            gt2 = v > m2
            m2 = jnp.where(gt1, m1, jnp.where(gt2, v, m2))
            i2 = jnp.where(gt1, i1, jnp.where(gt2, ev, i2))
            m1 = jnp.where(gt1, v, m1)
            i1 = jnp.where(gt1, ev, i1)

        one = jnp.ones((_L,), jnp.float32)
        for e in range(_NUM_EXPERTS):
            hits = (jnp.where(i1 == e, one, 0.0) +
                    jnp.where(i2 == e, one, 0.0))
            cntacc[e, :] = cntacc[e, :] + hits

        t = jnp.exp(m2 - m1)
        w1 = 1.0 / (1.0 + t)
        wstage[0, sl] = w1
        wstage[1, sl] = 1.0 - w1
        estage[0, sl] = i1
        estage[1, sl] = i2
        return 0

    lax.fori_loop(0, n_groups, group_body, 0)

    base = wid * chunk
    pltpu.sync_copy(wstage.at[0], w_hbm.at[0, pl.ds(base, chunk)])
    pltpu.sync_copy(wstage.at[1], w_hbm.at[1, pl.ds(base, chunk)])
    pltpu.sync_copy(estage.at[0], e_hbm.at[0, pl.ds(base, chunk)])
    pltpu.sync_copy(estage.at[1], e_hbm.at[1, pl.ds(base, chunk)])
    pltpu.sync_copy(cntacc, cnt_hbm.at[wid])


def _loss_body(ps_ref, cnt_ref, loss_out, *, num_tokens):
    cnt = jnp.sum(cnt_ref[...], axis=(0, 2))
    scale = _ALPHA * _NUM_EXPERTS / (num_tokens * num_tokens)
    loss_out[...] = scale * jnp.sum(ps_ref[0, :] * cnt, keepdims=True)[None]


def kernel(hidden_states, W_gate):
    batch, seq, hidden = hidden_states.shape
    num_tokens = batch * seq
    x = hidden_states.reshape(num_tokens, hidden)
    wt = W_gate.T
    chunk = num_tokens // _NW
    n_steps = num_tokens // _TILE
    sub = _TILE // 1024

    logits_t, psum = pl.pallas_call(
        functools.partial(_matmul_body, n_steps=n_steps),
        grid=(n_steps,),
        in_specs=[
            pl.BlockSpec((_TILE, hidden), lambda i: (i, 0)),
            pl.BlockSpec((hidden, _NUM_EXPERTS), lambda i: (0, 0)),
        ],
        out_specs=[
            pl.BlockSpec((sub, _NUM_EXPERTS, 1024), lambda i: (i, 0, 0)),
            pl.BlockSpec((1, _NUM_EXPERTS), lambda i: (0, 0)),
        ],
        out_shape=[
            jax.ShapeDtypeStruct((num_tokens // 1024, _NUM_EXPERTS, 1024),
                                 jnp.float32),
            jax.ShapeDtypeStruct((1, _NUM_EXPERTS), jnp.float32),
        ],
        scratch_shapes=[
            pltpu.VMEM((1, _NUM_EXPERTS), jnp.float32),
        ],
    )(x, wt)

    sc_router = pl.kernel(
        _sc_router_body,
        mesh=plsc.VectorSubcoreMesh(core_axis_name="c", subcore_axis_name="s"),
        out_type=[
            jax.ShapeDtypeStruct((2, num_tokens), jnp.float32),
            jax.ShapeDtypeStruct((2, num_tokens), jnp.int32),
            jax.ShapeDtypeStruct((_NW, _NUM_EXPERTS, _L), jnp.float32),
        ],
        scratch_types=[
            pltpu.VMEM((_NUM_EXPERTS, chunk), jnp.float32),
            pltpu.VMEM((2, chunk), jnp.float32),
            pltpu.VMEM((2, chunk), jnp.int32),
            pltpu.VMEM((_NUM_EXPERTS, _L), jnp.float32),
            pltpu.SemaphoreType.DMA,
        ],
    )
    w2, e2, cnt_part = sc_router(logits_t)

    loss = pl.pallas_call(
        functools.partial(_loss_body, num_tokens=num_tokens),
        in_specs=[
            pl.BlockSpec((1, _NUM_EXPERTS), lambda: (0, 0)),
            pl.BlockSpec((_NW, _NUM_EXPERTS, _L), lambda: (0, 0, 0)),
        ],
        out_specs=pl.BlockSpec((1, 1), lambda: (0, 0)),
        out_shape=jax.ShapeDtypeStruct((1, 1), jnp.float32),
    )(psum, cnt_part)

    weights = jnp.stack([w2[0], w2[1]], axis=-1)
    experts = jnp.stack([e2[0], e2[1]], axis=-1)
    return (weights.reshape(batch, seq, _TOP_K),
            experts.reshape(batch, seq, _TOP_K),
            loss[0, 0])


# speedup vs baseline: 1.0171x; 1.0171x over previous
import functools

import jax
import jax.numpy as jnp
from jax import lax
from jax.experimental import pallas as pl
from jax.experimental.pallas import tpu as pltpu
from jax.experimental.pallas import tpu_sc as plsc

_NUM_EXPERTS = 64
_TOP_K = 2
_ALPHA = 0.01
_TILE = 4096

_NC = 2
_NS = 16
_NW = _NC * _NS
_L = 16


def _matmul_body(x_ref, wt_ref, lt_out, ps_out, psacc, *, n_steps):
    pid = pl.program_id(0)

    @pl.when(pid == 0)
    def _init():
        psacc[...] = jnp.zeros_like(psacc)

    tile = x_ref.shape[0]
    logits = jnp.dot(x_ref[...], wt_ref[...],
                     preferred_element_type=jnp.float32)
    sub = tile // 1024
    lt_out[...] = logits.reshape(sub, 1024, _NUM_EXPERTS).transpose(0, 2, 1)

    m = jnp.max(logits, axis=-1, keepdims=True)
    e = jnp.exp(logits - m)
    z = jnp.sum(e, axis=-1, keepdims=True)
    psacc[...] += jnp.sum(e / z, axis=0, keepdims=True)

    @pl.when(pid == n_steps - 1)
    def _fin():
        ps_out[...] = psacc[...]


def _sc_router_body(lt_hbm, w_hbm, e_hbm, cnt_hbm,
                    blk, wstage, estage, cntacc, dsem):
    wid = lax.axis_index("s") * _NC + lax.axis_index("c")
    chunk = blk.shape[1]
    n_groups = chunk // _L

    cp = pltpu.async_copy(lt_hbm.at[wid], blk, dsem)

    zf = jnp.zeros((_L,), jnp.float32)

    def zero_body(e, _):
        cntacc[e, :] = zf
        return 0

    lax.fori_loop(0, _NUM_EXPERTS, zero_body, 0)
    cp.wait()

    def group_body(g, _):
        sl = pl.ds(g * _L, _L)

        m1 = blk[0, sl]
        i1 = jnp.zeros((_L,), jnp.int32)
        m2 = jnp.full((_L,), -jnp.inf, jnp.float32)
        i2 = jnp.zeros((_L,), jnp.int32)
        for e in range(1, _NUM_EXPERTS):
            v = blk[e, sl]
            ev = jnp.full((_L,), e, jnp.int32)
            gt1 = v > m1
            gt2 = v > m2
            m2 = jnp.where(gt1, m1, jnp.where(gt2, v, m2))
            i2 = jnp.where(gt1, i1, jnp.where(gt2, ev, i2))
            m1 = jnp.where(gt1, v, m1)
            i1 = jnp.where(gt1, ev, i1)

        one = jnp.ones((_L,), jnp.float32)
        for e in range(_NUM_EXPERTS):
            hits = (jnp.where(i1 == e, one, 0.0) +
                    jnp.where(i2 == e, one, 0.0))
            cntacc[e, :] = cntacc[e, :] + hits

        t = jnp.exp(m2 - m1)
        w1 = 1.0 / (1.0 + t)
        wstage[0, sl] = w1
        wstage[1, sl] = 1.0 - w1
        estage[0, sl] = i1
        estage[1, sl] = i2
        return 0

    lax.fori_loop(0, n_groups, group_body, 0)

    base = wid * chunk
    pltpu.sync_copy(wstage.at[0], w_hbm.at[0, pl.ds(base, chunk)])
    pltpu.sync_copy(wstage.at[1], w_hbm.at[1, pl.ds(base, chunk)])
    pltpu.sync_copy(estage.at[0], e_hbm.at[0, pl.ds(base, chunk)])
    pltpu.sync_copy(estage.at[1], e_hbm.at[1, pl.ds(base, chunk)])
    pltpu.sync_copy(cntacc, cnt_hbm.at[wid])


def _loss_body(ps_ref, cnt_ref, loss_out, *, num_tokens):
    cnt = jnp.sum(cnt_ref[...], axis=(0, 2))
    scale = _ALPHA * _NUM_EXPERTS / (num_tokens * num_tokens)
    loss_out[...] = scale * jnp.sum(ps_ref[0, :] * cnt, keepdims=True)[None]


def kernel(hidden_states, W_gate):
    batch, seq, hidden = hidden_states.shape
    num_tokens = batch * seq
    x = hidden_states.reshape(num_tokens, hidden)
    wt = W_gate.T
    chunk = num_tokens // _NW
    n_steps = num_tokens // _TILE
    sub = _TILE // 1024

    logits_t, psum = pl.pallas_call(
        functools.partial(_matmul_body, n_steps=n_steps),
        grid=(n_steps,),
        in_specs=[
            pl.BlockSpec((_TILE, hidden), lambda i: (i, 0)),
            pl.BlockSpec((hidden, _NUM_EXPERTS), lambda i: (0, 0)),
        ],
        out_specs=[
            pl.BlockSpec((sub, _NUM_EXPERTS, 1024), lambda i: (i, 0, 0)),
            pl.BlockSpec((1, _NUM_EXPERTS), lambda i: (0, 0)),
        ],
        out_shape=[
            jax.ShapeDtypeStruct((num_tokens // 1024, _NUM_EXPERTS, 1024),
                                 jnp.float32),
            jax.ShapeDtypeStruct((1, _NUM_EXPERTS), jnp.float32),
        ],
        scratch_shapes=[
            pltpu.VMEM((1, _NUM_EXPERTS), jnp.float32),
        ],
    )(x, wt)

    sc_router = pl.kernel(
        _sc_router_body,
        mesh=plsc.VectorSubcoreMesh(core_axis_name="c", subcore_axis_name="s"),
        out_type=[
            jax.ShapeDtypeStruct((2, num_tokens), jnp.float32),
            jax.ShapeDtypeStruct((2, num_tokens), jnp.int32),
            jax.ShapeDtypeStruct((_NW, _NUM_EXPERTS, _L), jnp.float32),
        ],
        scratch_types=[
            pltpu.VMEM((_NUM_EXPERTS, chunk), jnp.float32),
            pltpu.VMEM((2, chunk), jnp.float32),
            pltpu.VMEM((2, chunk), jnp.int32),
            pltpu.VMEM((_NUM_EXPERTS, _L), jnp.float32),
            pltpu.SemaphoreType.DMA,
        ],
    )
    w2, e2, cnt_part = sc_router(logits_t)

    loss = pl.pallas_call(
        functools.partial(_loss_body, num_tokens=num_tokens),
        in_specs=[
            pl.BlockSpec((1, _NUM_EXPERTS), lambda: (0, 0)),
            pl.BlockSpec((_NW, _NUM_EXPERTS, _L), lambda: (0, 0, 0)),
        ],
        out_specs=pl.BlockSpec((1, 1), lambda: (0, 0)),
        out_shape=jax.ShapeDtypeStruct((1, 1), jnp.float32),
    )(psum, cnt_part)

    weights = jnp.stack([w2[0], w2[1]], axis=-1)
    experts = jnp.stack([e2[0], e2[1]], axis=-1)
    return (weights.reshape(batch, seq, _TOP_K),
            experts.reshape(batch, seq, _TOP_K),
            loss[0, 0])
